# Initial kernel scaffold; baseline (speedup 1.0000x reference)
import jax
import jax.numpy as jnp
from jax.experimental import pallas as pl


def kernel(x, edge_index, edge_weights, states, a0, W1, b1, Wc, bc):
    raise NotImplementedError("write your pallas kernel here")


# speedup vs baseline: 4.0142x; 4.0142x over previous
import dataclasses
import functools

import jax
import jax.numpy as jnp
from jax import lax
from jax.experimental import pallas as pl
from jax.experimental.pallas import tpu as pltpu
from jax.experimental.pallas import tpu_sc as plsc

_NC = 2
_NS = 16
_LANES = 16

_CHUNK = 128
_ROW_BLK = 1000


def _dense_body(x_ref, st_ref, a0_ref, w1_ref, b1_ref, wc_ref, bc_ref, h_ref):
    xb = x_ref[...]
    h = xb + st_ref[...] * a0_ref[0, 0]
    t = lax.dot_general(h, w1_ref[...], (((1,), (1,)), ((), ())),
                        preferred_element_type=jnp.float32,
                        precision=lax.Precision.HIGHEST)
    t = t + b1_ref[...]
    t = jnp.where(t >= 0, t, 0.2 * t)
    o = lax.dot_general(t, wc_ref[...], (((1,), (1,)), ((), ())),
                        preferred_element_type=jnp.float32,
                        precision=lax.Precision.HIGHEST)
    h_ref[...] = o + bc_ref[...]


def _dense(x, states2d, a0, W1, b1r, Wc, bcr):
    n, d = x.shape
    grid = n // _ROW_BLK
    return pl.pallas_call(
        _dense_body,
        grid=(grid,),
        in_specs=[
            pl.BlockSpec((_ROW_BLK, d), lambda i: (i, 0)),
            pl.BlockSpec((_ROW_BLK, 1), lambda i: (i, 0)),
            pl.BlockSpec((1, 1), lambda i: (0, 0)),
            pl.BlockSpec((d, d), lambda i: (0, 0)),
            pl.BlockSpec((1, d), lambda i: (0, 0)),
            pl.BlockSpec((d, d), lambda i: (0, 0)),
            pl.BlockSpec((1, d), lambda i: (0, 0)),
        ],
        out_specs=pl.BlockSpec((_ROW_BLK, d), lambda i: (i, 0)),
        out_shape=jax.ShapeDtypeStruct((n, d), jnp.float32),
    )(x, states2d, a0, W1, b1r, Wc, bcr)


def _make_sc_edge_kernel(n_pad, e, d):
    total_chunks = e // _CHUNK
    nw = _NC * _NS
    t_max = (total_chunks + nw - 1) // nw
    rows_per_sub = n_pad // _NS
    mesh = plsc.VectorSubcoreMesh(core_axis_name="c", subcore_axis_name="s")
    cp = pltpu.CompilerParams()
    if "needs_layout_passes" in pltpu.CompilerParams.__dataclass_fields__:
        cp = dataclasses.replace(cp, needs_layout_passes=False)

    @functools.partial(
        pl.kernel,
        out_type=jax.ShapeDtypeStruct((_NC, n_pad, d), jnp.float32),
        mesh=mesh,
        scratch_types=[
            pltpu.VMEM((_CHUNK,), jnp.int32),
            pltpu.VMEM((_CHUNK,), jnp.int32),
            pltpu.VMEM((_CHUNK,), jnp.float32),
            pltpu.VMEM((_CHUNK, d), jnp.float32),
            pltpu.VMEM_SHARED((n_pad, d), jnp.float32),
            pltpu.SemaphoreType.DMA,
        ],
        compiler_params=cp,
    )
    def sc_kernel(h_hbm, src_hbm, dst_hbm, w_hbm, zeros_hbm, out_hbm,
                  src_v, dst_v, w_v, rows_v, acc, sem):
        c = lax.axis_index("c")
        s = lax.axis_index("s")
        wid = s * _NC + c

        pltpu.sync_copy(zeros_hbm.at[pl.ds(s * rows_per_sub, rows_per_sub)],
                        acc.at[pl.ds(s * rows_per_sub, rows_per_sub)])
        plsc.subcore_barrier()

        @pl.loop(0, t_max)
        def _chunk_loop(t):
            cid = wid + t * nw

            @pl.when(cid < total_chunks)
            def _():
                base = cid * _CHUNK
                pltpu.sync_copy(src_hbm.at[pl.ds(base, _CHUNK)], src_v)
                pltpu.sync_copy(dst_hbm.at[pl.ds(base, _CHUNK)], dst_v)
                pltpu.sync_copy(w_hbm.at[pl.ds(base, _CHUNK)], w_v)
                pltpu.async_copy(h_hbm.at[src_v], rows_v, sem).wait()

                @pl.loop(0, _CHUNK)
                def _scale(ei):
                    idx = jnp.broadcast_to(ei, (_LANES,)).astype(jnp.int32)
                    wsplat = plsc.load_gather(w_v, [idx])
                    for seg in range(d // _LANES):
                        sl = pl.ds(seg * _LANES, _LANES)
                        rows_v[ei, sl] = rows_v[ei, sl] * wsplat

                pltpu.sync_copy(rows_v, acc.at[dst_v], add=True)

        plsc.subcore_barrier()
        pltpu.sync_copy(acc.at[pl.ds(s * rows_per_sub, rows_per_sub)],
                        out_hbm.at[c, pl.ds(s * rows_per_sub, rows_per_sub)])

    return sc_kernel


def _combine_body(a_ref, b_ref, o_ref):
    o_ref[...] = a_ref[0] + b_ref[0]


def _combine(partial, n, d):
    grid = n // _ROW_BLK
    return pl.pallas_call(
        _combine_body,
        grid=(grid,),
        in_specs=[
            pl.BlockSpec((1, _ROW_BLK, d), lambda i: (0, i, 0)),
            pl.BlockSpec((1, _ROW_BLK, d), lambda i: (1, i, 0)),
        ],
        out_specs=pl.BlockSpec((_ROW_BLK, d), lambda i: (i, 0)),
        out_shape=jax.ShapeDtypeStruct((n, d), jnp.float32),
    )(partial, partial)


def kernel(x, edge_index, edge_weights, states, a0, W1, b1, Wc, bc):
    n, d = x.shape
    e = edge_weights.shape[0]
    n_pad = ((n + 8 * _NS - 1) // (8 * _NS)) * (8 * _NS)
    assert e % _CHUNK == 0 and n % _ROW_BLK == 0

    h = _dense(x, states[:, None], a0, W1, b1[None, :], Wc, bc[None, :])

    src = edge_index[0]
    dst = edge_index[1]
    zeros = jnp.zeros((n_pad, d), jnp.float32)
    sc_kernel = _make_sc_edge_kernel(n_pad, e, d)
    partial = sc_kernel(h, src, dst, edge_weights, zeros)

    return _combine(partial, n, d)
